# baseline (speedup 1.0000x reference)
import functools

import jax
import jax.numpy as jnp
from jax.experimental import pallas as pl
from jax.experimental.pallas import tpu as pltpu

_THRESHOLD = 0.03


def _moe_body(x_ref, wg_ref, bg_ref, we_ref, be_ref, out_ref, w_scr, *, n_experts):
    e = pl.program_id(1)

    @pl.when(e == 0)
    def _compute_gate():
        logits = (
            jnp.dot(x_ref[...], wg_ref[...], preferred_element_type=jnp.float32)
            + bg_ref[...]
        )
        mx = jnp.max(logits, axis=-1, keepdims=True)
        p = jnp.exp(logits - mx)
        p = p / jnp.sum(p, axis=-1, keepdims=True)
        sel = jnp.where(p >= _THRESHOLD, p, 0.0)
        s = jnp.sum(sel, axis=-1, keepdims=True)
        s = jnp.where(s == 0.0, 1.0, s)
        w_scr[...] = sel / s

    expert_out = (
        jnp.dot(x_ref[...], we_ref[0], preferred_element_type=jnp.float32)
        + be_ref[0]
    )
    onehot = (
        jax.lax.broadcasted_iota(jnp.int32, (1, n_experts), 1) == e
    ).astype(jnp.float32)
    w_col = jnp.sum(w_scr[...] * onehot, axis=-1, keepdims=True)
    contrib = w_col * expert_out

    @pl.when(e == 0)
    def _init():
        out_ref[...] = contrib

    @pl.when(e != 0)
    def _accum():
        out_ref[...] += contrib


def kernel(inputs, Wg, bg, We, be):
    D = inputs.shape[-1]
    E = We.shape[0]
    flat = inputs.reshape(-1, D)
    N = flat.shape[0]
    TM = min(2048, N)
    nm = N // TM

    out = pl.pallas_call(
        functools.partial(_moe_body, n_experts=E),
        grid=(nm, E),
        in_specs=[
            pl.BlockSpec((TM, D), lambda m, e: (m, 0)),
            pl.BlockSpec((D, E), lambda m, e: (0, 0)),
            pl.BlockSpec((1, E), lambda m, e: (0, 0)),
            pl.BlockSpec((1, D, D), lambda m, e: (e, 0, 0)),
            pl.BlockSpec((1, 1, D), lambda m, e: (e, 0, 0)),
        ],
        out_specs=pl.BlockSpec((TM, D), lambda m, e: (m, 0)),
        out_shape=jax.ShapeDtypeStruct((N, D), jnp.float32),
        scratch_shapes=[pltpu.VMEM((TM, E), jnp.float32)],
        compiler_params=pltpu.CompilerParams(
            dimension_semantics=("parallel", "arbitrary"),
        ),
    )(flat, Wg, bg.reshape(1, E), We, be.reshape(E, 1, D))
    return out.reshape(inputs.shape[:-1] + (D,))


# speedup vs baseline: 1.1320x; 1.1320x over previous
import functools

import jax
import jax.numpy as jnp
from jax.experimental import pallas as pl
from jax.experimental.pallas import tpu as pltpu

_THRESHOLD = 0.03


def _moe_body(x_ref, wg_ref, bg_ref, we_ref, be_ref, out_ref, w_scr, *, n_experts):
    e = pl.program_id(1)

    @pl.when(e == 0)
    def _compute_gate():
        logits = (
            jnp.dot(x_ref[...], wg_ref[...], preferred_element_type=jnp.float32)
            + bg_ref[...]
        )
        mx = jnp.max(logits, axis=-1, keepdims=True)
        p = jnp.exp(logits - mx)
        p = p / jnp.sum(p, axis=-1, keepdims=True)
        sel = jnp.where(p >= _THRESHOLD, p, 0.0)
        s = jnp.sum(sel, axis=-1, keepdims=True)
        s = jnp.where(s == 0.0, 1.0, s)
        w_scr[...] = sel / s
        out_ref[...] = jnp.zeros_like(out_ref)

    expert_out = (
        jnp.dot(x_ref[...], we_ref[0], preferred_element_type=jnp.float32)
        + be_ref[0]
    )
    onehot = (
        jax.lax.broadcasted_iota(jnp.int32, (1, n_experts), 1) == e
    ).astype(jnp.float32)
    w_col = jnp.sum(w_scr[...] * onehot, axis=-1, keepdims=True)
    out_ref[...] += w_col * expert_out


def kernel(inputs, Wg, bg, We, be):
    D = inputs.shape[-1]
    E = We.shape[0]
    flat = inputs.reshape(-1, D)
    N = flat.shape[0]
    TM = min(2048, N)
    nm = N // TM

    out = pl.pallas_call(
        functools.partial(_moe_body, n_experts=E),
        grid=(nm, E),
        in_specs=[
            pl.BlockSpec((TM, D), lambda m, e: (m, 0)),
            pl.BlockSpec((D, E), lambda m, e: (0, 0)),
            pl.BlockSpec((1, E), lambda m, e: (0, 0)),
            pl.BlockSpec((1, D, D), lambda m, e: (e, 0, 0)),
            pl.BlockSpec((1, 1, D), lambda m, e: (e, 0, 0)),
        ],
        out_specs=pl.BlockSpec((TM, D), lambda m, e: (m, 0)),
        out_shape=jax.ShapeDtypeStruct((N, D), jnp.float32),
        scratch_shapes=[pltpu.VMEM((TM, E), jnp.float32)],
        compiler_params=pltpu.CompilerParams(
            dimension_semantics=("parallel", "arbitrary"),
        ),
    )(flat, Wg, bg.reshape(1, E), We, be.reshape(E, 1, D))
    return out.reshape(inputs.shape[:-1] + (D,))
